# Initial kernel scaffold; baseline (speedup 1.0000x reference)
import jax
import jax.numpy as jnp
from jax.experimental import pallas as pl


def kernel(outputs_feat, styles_feat):
    raise NotImplementedError("write your pallas kernel here")


# speedup vs baseline: 4.8942x; 4.8942x over previous
import jax
import jax.numpy as jnp
from jax.experimental import pallas as pl
from jax.experimental.pallas import tpu as pltpu

_C = 256
_HW = 4096
_BJ = 512
_NJ = _HW // _BJ


def _nnfm_loss_kernel(a_ref, b_ref, out_ref, rmax_ref):
    j = pl.program_id(0)
    a = a_ref[...]
    b = b_ref[...]
    b_norm = jnp.sqrt(jnp.sum(b * b, axis=0, keepdims=True))
    b_inv = 1.0 / (b_norm + 1e-8)
    m = jax.lax.dot_general(
        a.astype(jnp.bfloat16), b.astype(jnp.bfloat16),
        (((0,), (0,)), ((), ())),
        preferred_element_type=jnp.float32)
    pmax = jnp.max(m * b_inv, axis=1, keepdims=True)

    @pl.when(j == 0)
    def _init():
        rmax_ref[...] = pmax

    @pl.when(j > 0)
    def _acc():
        rmax_ref[...] = jnp.maximum(rmax_ref[...], pmax)

    @pl.when(j == _NJ - 1)
    def _finish():
        a_norm = jnp.sqrt(jnp.sum(a * a, axis=0, keepdims=True))
        a_inv = 1.0 / (a_norm + 1e-8)
        s = jax.lax.dot_general(
            a_inv, rmax_ref[...], (((1,), (0,)), ((), ())),
            preferred_element_type=jnp.float32)
        out_ref[...] = 1.0 - s * (1.0 / _HW)


def kernel(outputs_feat, styles_feat):
    a = outputs_feat.reshape(_C, _HW)
    b = styles_feat.reshape(_C, _HW)
    out = pl.pallas_call(
        _nnfm_loss_kernel,
        grid=(_NJ,),
        in_specs=[
            pl.BlockSpec((_C, _HW), lambda j: (0, 0)),
            pl.BlockSpec((_C, _BJ), lambda j: (0, j)),
        ],
        out_specs=pl.BlockSpec((1, 1), lambda j: (0, 0)),
        out_shape=jax.ShapeDtypeStruct((1, 1), jnp.float32),
        scratch_shapes=[pltpu.VMEM((_HW, 1), jnp.float32)],
    )(a, b)
    return out[0, 0]
